# baseline (speedup 1.0000x reference)
import jax
import jax.numpy as jnp
from jax.experimental import pallas as pl
from jax.experimental.pallas import tpu as pltpu

_B = 64
_NC = 64
_NS = 128
_F = 128
_K = 10
_BB = 2
_GRID = _B // _BB


def _extreme_sum(v):
    t0 = jnp.max(v, axis=(1, 2), keepdims=True)
    c0 = jnp.sum((v >= t0).astype(jnp.float32), axis=(1, 2), keepdims=True)
    done0 = (c0 >= _K).astype(jnp.float32)

    def it(_, carry):
        t, done = carry
        m = jnp.max(jnp.where(v < t, v, -jnp.inf), axis=(1, 2), keepdims=True)
        t_new = jnp.where(done > 0.5, t, m)
        cnt = jnp.sum((v >= t_new).astype(jnp.float32), axis=(1, 2), keepdims=True)
        return t_new, (cnt >= _K).astype(jnp.float32)

    t, _ = jax.lax.fori_loop(0, _K - 1, it, (t0, done0))
    above = jnp.sum(jnp.where(v > t, v, 0.0), axis=(1, 2), keepdims=True)
    cnt_above = jnp.sum((v > t).astype(jnp.float32), axis=(1, 2), keepdims=True)
    return above + t * (_K - cnt_above)


def _body(x_ref, w_ref, b_ref, out_ref):
    xb = x_ref[...]
    w = w_ref[0, :]
    s = jnp.sum(xb * w[None, None, None, :], axis=3)
    top = _extreme_sum(s)
    bot = -_extreme_sum(-s)
    out_ref[...] = ((top + bot) / (2.0 * _K) + b_ref[0, 0]).reshape(1, _BB, 1)


def kernel(x, W, b):
    b2 = b.reshape(1, 1).astype(jnp.float32)
    x4 = x.reshape(_B, _NC, _NS, _F)
    out = pl.pallas_call(
        _body,
        grid=(_GRID,),
        in_specs=[
            pl.BlockSpec((_BB, _NC, _NS, _F), lambda i: (i, 0, 0, 0)),
            pl.BlockSpec((1, _F), lambda i: (0, 0)),
            pl.BlockSpec(memory_space=pltpu.SMEM),
        ],
        out_specs=pl.BlockSpec((1, _BB, 1), lambda i: (i, 0, 0)),
        out_shape=jax.ShapeDtypeStruct((_GRID, _BB, 1), jnp.float32),
    )(x4, W, b2)
    return out.reshape(_B, 1)


# speedup vs baseline: 9.4925x; 9.4925x over previous
import jax
import jax.numpy as jnp
from jax.experimental import pallas as pl
from jax.experimental.pallas import tpu as pltpu

_B = 64
_NC = 64
_NS = 128
_F = 128
_K = 10
_BB = 2
_GRID = _B // _BB


def _extreme_sum(v):
    t0 = jnp.max(v, axis=(1, 2), keepdims=True)
    c0 = jnp.sum((v >= t0).astype(jnp.float32), axis=(1, 2), keepdims=True)
    done0 = (c0 >= _K).astype(jnp.float32)

    def it(_, carry):
        t, done = carry
        m = jnp.max(jnp.where(v < t, v, -jnp.inf), axis=(1, 2), keepdims=True)
        t_new = jnp.where(done > 0.5, t, m)
        cnt = jnp.sum((v >= t_new).astype(jnp.float32), axis=(1, 2), keepdims=True)
        return t_new, (cnt >= _K).astype(jnp.float32)

    t, _ = jax.lax.fori_loop(0, _K - 1, it, (t0, done0))
    above = jnp.sum(jnp.where(v > t, v, 0.0), axis=(1, 2), keepdims=True)
    cnt_above = jnp.sum((v > t).astype(jnp.float32), axis=(1, 2), keepdims=True)
    return above + t * (_K - cnt_above)


def _body(x_ref, w_ref, b_ref, out_ref, s_ref):
    xb = x_ref[...]
    w = w_ref[0, :]
    s_ref[...] = jnp.sum(xb * w[None, None, None, :], axis=3)
    s = s_ref[...]
    top = _extreme_sum(s)
    bot = -_extreme_sum(-s)
    out_ref[...] = ((top + bot) / (2.0 * _K) + b_ref[0, 0]).reshape(1, _BB, 1)


def kernel(x, W, b):
    b2 = b.reshape(1, 1).astype(jnp.float32)
    x4 = x.reshape(_B, _NC, _NS, _F)
    out = pl.pallas_call(
        _body,
        grid=(_GRID,),
        in_specs=[
            pl.BlockSpec((_BB, _NC, _NS, _F), lambda i: (i, 0, 0, 0)),
            pl.BlockSpec((1, _F), lambda i: (0, 0)),
            pl.BlockSpec(memory_space=pltpu.SMEM),
        ],
        out_specs=pl.BlockSpec((1, _BB, 1), lambda i: (i, 0, 0)),
        out_shape=jax.ShapeDtypeStruct((_GRID, _BB, 1), jnp.float32),
        scratch_shapes=[pltpu.VMEM((_BB, _NC, _NS), jnp.float32)],
    )(x4, W, b2)
    return out.reshape(_B, 1)


# speedup vs baseline: 18.2293x; 1.9204x over previous
import jax
import jax.numpy as jnp
from jax.experimental import pallas as pl
from jax.experimental.pallas import tpu as pltpu

_B = 64
_NC = 64
_NS = 128
_F = 128
_K = 10
_BB = 4
_GRID = _B // _BB


def _extreme_sum(v):
    t0 = jnp.max(v, axis=(1, 2), keepdims=True)
    c0 = jnp.sum((v >= t0).astype(jnp.float32), axis=(1, 2), keepdims=True)
    done0 = (c0 >= _K).astype(jnp.float32)

    def it(_, carry):
        t, done = carry
        m = jnp.max(jnp.where(v < t, v, -jnp.inf), axis=(1, 2), keepdims=True)
        t_new = jnp.where(done > 0.5, t, m)
        cnt = jnp.sum((v >= t_new).astype(jnp.float32), axis=(1, 2), keepdims=True)
        return t_new, (cnt >= _K).astype(jnp.float32)

    t, _ = jax.lax.fori_loop(0, _K - 1, it, (t0, done0))
    above = jnp.sum(jnp.where(v > t, v, 0.0), axis=(1, 2), keepdims=True)
    cnt_above = jnp.sum((v > t).astype(jnp.float32), axis=(1, 2), keepdims=True)
    return above + t * (_K - cnt_above)


def _body(x_ref, w_ref, b_ref, out_ref, s_ref):
    i = pl.program_id(0)
    xb = x_ref[...]
    w = w_ref[0, :]
    s = jnp.sum(xb * w[None, None, None, :], axis=3)
    s_ref[pl.ds(i * _BB, _BB), :, :] = s

    @pl.when(i == _GRID - 1)
    def _():
        sc = s_ref[...]
        top = _extreme_sum(sc)
        bot = -_extreme_sum(-sc)
        out_ref[...] = ((top + bot) / (2.0 * _K) + b_ref[0, 0]).reshape(_B, 1)


def kernel(x, W, b):
    b2 = b.reshape(1, 1).astype(jnp.float32)
    x4 = x.reshape(_B, _NC, _NS, _F)
    out = pl.pallas_call(
        _body,
        grid=(_GRID,),
        in_specs=[
            pl.BlockSpec((_BB, _NC, _NS, _F), lambda i: (i, 0, 0, 0)),
            pl.BlockSpec((1, _F), lambda i: (0, 0)),
            pl.BlockSpec(memory_space=pltpu.SMEM),
        ],
        out_specs=pl.BlockSpec((_B, 1), lambda i: (0, 0)),
        out_shape=jax.ShapeDtypeStruct((_B, 1), jnp.float32),
        scratch_shapes=[pltpu.VMEM((_B, _NC, _NS), jnp.float32)],
    )(x4, W, b2)
    return out
